# Initial kernel scaffold; baseline (speedup 1.0000x reference)
import jax
import jax.numpy as jnp
from jax.experimental import pallas as pl


def kernel(x, weight):
    raise NotImplementedError("write your pallas kernel here")


# speedup vs baseline: 6.8884x; 6.8884x over previous
import functools

import jax
import jax.numpy as jnp
from jax.experimental import pallas as pl

NUM_EXPERTS = 64
TOPK = 8
TOKEN_BLOCK = 256


def _router_kernel(x_ref, w_ref, probs_ref, map_ref):
    logits = jax.lax.dot_general(
        x_ref[...], w_ref[...],
        dimension_numbers=(((1,), (1,)), ((), ())),
        preferred_element_type=jnp.float32,
    )
    row_max = jnp.max(logits, axis=1, keepdims=True)
    e = jnp.exp(logits - row_max)
    p = e / jnp.sum(e, axis=1, keepdims=True)

    cols = jax.lax.broadcasted_iota(jnp.int32, logits.shape, 1)
    v = logits
    mask = jnp.zeros(logits.shape, dtype=jnp.bool_)
    neg_inf = jnp.float32(-jnp.inf)
    for _ in range(TOPK):
        mx = jnp.max(v, axis=1, keepdims=True)
        at_max = v == mx
        first = jnp.min(jnp.where(at_max, cols, NUM_EXPERTS), axis=1, keepdims=True)
        sel = cols == first
        mask = jnp.logical_or(mask, sel)
        v = jnp.where(sel, neg_inf, v)

    probs_ref[...] = jnp.where(mask, p, 0.0)
    map_ref[...] = mask.astype(jnp.int8)


@jax.jit
def kernel(x, weight):
    num_tokens, hidden = x.shape
    grid = (num_tokens // TOKEN_BLOCK,)
    probs, rmap = pl.pallas_call(
        _router_kernel,
        grid=grid,
        in_specs=[
            pl.BlockSpec((TOKEN_BLOCK, hidden), lambda i: (i, 0)),
            pl.BlockSpec((NUM_EXPERTS, hidden), lambda i: (0, 0)),
        ],
        out_specs=[
            pl.BlockSpec((TOKEN_BLOCK, NUM_EXPERTS), lambda i: (i, 0)),
            pl.BlockSpec((TOKEN_BLOCK, NUM_EXPERTS), lambda i: (i, 0)),
        ],
        out_shape=[
            jax.ShapeDtypeStruct((num_tokens, NUM_EXPERTS), jnp.float32),
            jax.ShapeDtypeStruct((num_tokens, NUM_EXPERTS), jnp.int8),
        ],
    )(x, weight)
    return probs, rmap.astype(jnp.bool_)


# speedup vs baseline: 9.8038x; 1.4232x over previous
import jax
import jax.numpy as jnp
from jax.experimental import pallas as pl

NUM_EXPERTS = 64
TOPK = 8
TOKEN_BLOCK = 256


def _router_kernel(x_ref, w_ref, probs_ref, map_ref):
    logits = jax.lax.dot_general(
        w_ref[...], x_ref[...],
        dimension_numbers=(((1,), (1,)), ((), ())),
        preferred_element_type=jnp.float32,
    )
    col_max = jnp.max(logits, axis=0, keepdims=True)
    e = jnp.exp(logits - col_max)
    p = e / jnp.sum(e, axis=0, keepdims=True)

    rows = jax.lax.broadcasted_iota(jnp.int32, logits.shape, 0)
    v = logits
    mask = jnp.zeros(logits.shape, dtype=jnp.bool_)
    neg_inf = jnp.float32(-jnp.inf)
    for _ in range(TOPK):
        mx = jnp.max(v, axis=0, keepdims=True)
        at_max = v == mx
        first = jnp.min(jnp.where(at_max, rows, NUM_EXPERTS), axis=0, keepdims=True)
        sel = rows == first
        mask = jnp.logical_or(mask, sel)
        v = jnp.where(sel, neg_inf, v)

    probs_ref[...] = jnp.where(mask, p, 0.0).T
    map_ref[...] = mask.astype(jnp.float32).T


@jax.jit
def kernel(x, weight):
    num_tokens, hidden = x.shape
    grid = (num_tokens // TOKEN_BLOCK,)
    probs, rmap = pl.pallas_call(
        _router_kernel,
        grid=grid,
        in_specs=[
            pl.BlockSpec((TOKEN_BLOCK, hidden), lambda i: (i, 0)),
            pl.BlockSpec((NUM_EXPERTS, hidden), lambda i: (0, 0)),
        ],
        out_specs=[
            pl.BlockSpec((TOKEN_BLOCK, NUM_EXPERTS), lambda i: (i, 0)),
            pl.BlockSpec((TOKEN_BLOCK, NUM_EXPERTS), lambda i: (i, 0)),
        ],
        out_shape=[
            jax.ShapeDtypeStruct((num_tokens, NUM_EXPERTS), jnp.float32),
            jax.ShapeDtypeStruct((num_tokens, NUM_EXPERTS), jnp.float32),
        ],
    )(x, weight)
    return probs, rmap.astype(jnp.bool_)


# speedup vs baseline: 11.5051x; 1.1735x over previous
import jax
import jax.numpy as jnp
from jax.experimental import pallas as pl

NUM_EXPERTS = 64
TOPK = 8
TOKEN_BLOCK = 512


def _router_kernel(x_ref, w_ref, probs_ref, map_ref):
    logits = jax.lax.dot_general(
        w_ref[...], x_ref[...],
        dimension_numbers=(((1,), (1,)), ((), ())),
        preferred_element_type=jnp.float32,
    )
    col_max = jnp.max(logits, axis=0, keepdims=True)
    e = jnp.exp(logits - col_max)
    p = e / jnp.sum(e, axis=0, keepdims=True)

    neg_inf = jnp.float32(-jnp.inf)

    v = logits
    for _ in range(TOPK):
        mx = jnp.max(v, axis=0, keepdims=True)
        v = jnp.where(v == mx, neg_inf, v)
    mask = v != logits
    cnt = jnp.sum(mask.astype(jnp.float32))
    exact = cnt == jnp.float32(TOPK * TOKEN_BLOCK)

    @pl.when(exact)
    def _():
        probs_ref[...] = jnp.where(mask, p, 0.0).T
        map_ref[...] = mask.astype(jnp.float32).T

    @pl.when(jnp.logical_not(exact))
    def _():
        rows = jax.lax.broadcasted_iota(jnp.int32, logits.shape, 0)
        v2 = logits
        m2 = jnp.zeros(logits.shape, dtype=jnp.bool_)
        for _ in range(TOPK):
            mx = jnp.max(v2, axis=0, keepdims=True)
            at_max = v2 == mx
            first = jnp.min(
                jnp.where(at_max, rows, NUM_EXPERTS), axis=0, keepdims=True
            )
            sel = rows == first
            m2 = jnp.logical_or(m2, sel)
            v2 = jnp.where(sel, neg_inf, v2)
        probs_ref[...] = jnp.where(m2, p, 0.0).T
        map_ref[...] = m2.astype(jnp.float32).T


@jax.jit
def kernel(x, weight):
    num_tokens, hidden = x.shape
    grid = (num_tokens // TOKEN_BLOCK,)
    probs, rmap = pl.pallas_call(
        _router_kernel,
        grid=grid,
        in_specs=[
            pl.BlockSpec((TOKEN_BLOCK, hidden), lambda i: (i, 0)),
            pl.BlockSpec((NUM_EXPERTS, hidden), lambda i: (0, 0)),
        ],
        out_specs=[
            pl.BlockSpec((TOKEN_BLOCK, NUM_EXPERTS), lambda i: (i, 0)),
            pl.BlockSpec((TOKEN_BLOCK, NUM_EXPERTS), lambda i: (i, 0)),
        ],
        out_shape=[
            jax.ShapeDtypeStruct((num_tokens, NUM_EXPERTS), jnp.float32),
            jax.ShapeDtypeStruct((num_tokens, NUM_EXPERTS), jnp.float32),
        ],
    )(x, weight)
    return probs, rmap.astype(jnp.bool_)


# speedup vs baseline: 11.5215x; 1.0014x over previous
import jax
import jax.numpy as jnp
from jax.experimental import pallas as pl

NUM_EXPERTS = 64
TOPK = 8
TOKEN_BLOCK = 512


def _router_kernel(x_ref, w_ref, probs_ref, map_ref):
    logits = jax.lax.dot_general(
        w_ref[...], x_ref[...],
        dimension_numbers=(((1,), (1,)), ((), ())),
        preferred_element_type=jnp.float32,
    )
    col_max = jnp.max(logits, axis=0, keepdims=True)
    e = jnp.exp(logits - col_max)
    p = e / jnp.sum(e, axis=0, keepdims=True)

    neg_inf = jnp.float32(-jnp.inf)

    v = p
    for _ in range(TOPK):
        mx = jnp.max(v, axis=0, keepdims=True)
        v = jnp.where(v == mx, neg_inf, v)
    mask = v != p
    cnt = jnp.sum(mask.astype(jnp.float32))
    exact = cnt == jnp.float32(TOPK * TOKEN_BLOCK)

    @pl.when(exact)
    def _():
        probs_ref[...] = jnp.where(mask, p, 0.0).T
        map_ref[...] = mask.astype(jnp.float32).T

    @pl.when(jnp.logical_not(exact))
    def _():
        rows = jax.lax.broadcasted_iota(jnp.int32, logits.shape, 0)
        v2 = p
        m2 = jnp.zeros(logits.shape, dtype=jnp.bool_)
        for _ in range(TOPK):
            mx = jnp.max(v2, axis=0, keepdims=True)
            at_max = v2 == mx
            first = jnp.min(
                jnp.where(at_max, rows, NUM_EXPERTS), axis=0, keepdims=True
            )
            sel = rows == first
            m2 = jnp.logical_or(m2, sel)
            v2 = jnp.where(sel, neg_inf, v2)
        probs_ref[...] = jnp.where(m2, p, 0.0).T
        map_ref[...] = m2.astype(jnp.float32).T


@jax.jit
def kernel(x, weight):
    num_tokens, hidden = x.shape
    grid = (num_tokens // TOKEN_BLOCK,)
    probs, rmap = pl.pallas_call(
        _router_kernel,
        grid=grid,
        in_specs=[
            pl.BlockSpec((TOKEN_BLOCK, hidden), lambda i: (i, 0)),
            pl.BlockSpec((NUM_EXPERTS, hidden), lambda i: (0, 0)),
        ],
        out_specs=[
            pl.BlockSpec((TOKEN_BLOCK, NUM_EXPERTS), lambda i: (i, 0)),
            pl.BlockSpec((TOKEN_BLOCK, NUM_EXPERTS), lambda i: (i, 0)),
        ],
        out_shape=[
            jax.ShapeDtypeStruct((num_tokens, NUM_EXPERTS), jnp.float32),
            jax.ShapeDtypeStruct((num_tokens, NUM_EXPERTS), jnp.float32),
        ],
    )(x, weight)
    return probs, rmap.astype(jnp.bool_)
